# baseline (speedup 1.0000x reference)
import functools

import jax
import jax.numpy as jnp
from jax import lax
from jax.experimental import pallas as pl
from jax.experimental.pallas import tpu as pltpu
from jax.experimental.pallas import tpu_sc as plsc

_NUM_CORES = 2
_NUM_SUBCORES = 16


def _proj_kernel(xp_ref, wa_ref, wb_ref, xs_ref, xe_ref):
    xv = xp_ref[...]
    xs_ref[...] = jnp.dot(
        xv, wa_ref[...], preferred_element_type=jnp.float32).astype(jnp.bfloat16)
    xe_ref[...] = jnp.dot(
        xv, wb_ref[...], preferred_element_type=jnp.float32).astype(jnp.bfloat16)


def _make_gather_add(E, H, slab_rows, slab_base):
    nw = _NUM_CORES * _NUM_SUBCORES
    half = E // 2
    rpw = slab_rows // nw
    unit = 200
    nu_side = rpw // unit
    nu = 2 * nu_side
    la = 2
    nbuf = 2 * la + 1

    mesh = plsc.VectorSubcoreMesh(
        core_axis_name="c", subcore_axis_name="s",
        num_cores=_NUM_CORES, num_subcores=_NUM_SUBCORES)

    @functools.partial(
        pl.kernel,
        out_type=jax.ShapeDtypeStruct((slab_rows, 2 * H), jnp.bfloat16),
        mesh=mesh,
        scratch_types=[
            pltpu.VMEM((2 * rpw,), jnp.int32),
            pltpu.VMEM((2 * rpw,), jnp.int32),
        ] + [pltpu.VMEM((unit, H), jnp.bfloat16) for _ in range(nbuf)]
          + [pltpu.SemaphoreType.DMA for _ in range(3 * nbuf + 1)],
        compiler_params=pltpu.CompilerParams(use_tc_tiling_on_sc=False),
    )
    def gather_add(eidx_hbm, xs_hbm, xe_hbm, out_hbm, idx_s, idx_e, *bufs):
        rows = bufs[:nbuf]
        isem = bufs[nbuf]
        bsem = bufs[nbuf + 1:2 * nbuf + 1]
        asem = bufs[2 * nbuf + 1:3 * nbuf + 1]
        ssem = bufs[3 * nbuf + 1:]
        wid = lax.axis_index("s") * _NUM_CORES + lax.axis_index("c")
        base = wid * rpw
        pre = []
        for row, dst in ((0, idx_s), (1, idx_e)):
            for side in range(2):
                e0 = side * half + slab_base + base
                pre.append(pltpu.async_copy(
                    eidx_hbm.at[row, pl.ds(e0, rpw)],
                    dst.at[pl.ds(side * rpw, rpw)], isem))
        for p in pre:
            p.wait()

        def src(u):
            side, j = u // nu_side, u % nu_side
            return side * rpw + j * unit

        def dst_rows(u):
            side, j = u // nu_side, u % nu_side
            return base + j * unit, side

        base_d = [None] * nbuf
        add_d = [None] * nbuf
        sc_d = [None] * nbuf
        for u in range(nu + 2 * la):
            if u < nu:
                b = u % nbuf
                if sc_d[b] is not None:
                    sc_d[b].wait()
                io = src(u)
                base_d[b] = pltpu.async_copy(
                    xs_hbm.at[idx_s.at[pl.ds(io, unit)]], rows[b], bsem[b])
            if la <= u < nu + la:
                b1 = (u - la) % nbuf
                base_d[b1].wait()
                io = src(u - la)
                add_d[b1] = pltpu.async_copy(
                    xe_hbm.at[idx_e.at[pl.ds(io, unit)]], rows[b1],
                    asem[b1], add=True)
            if u >= 2 * la:
                b2 = (u - 2 * la) % nbuf
                add_d[b2].wait()
                row, side = dst_rows(u - 2 * la)
                sc_d[b2] = pltpu.async_copy(
                    rows[b2],
                    out_hbm.at[pl.ds(row, unit), pl.ds(side * H, H)],
                    ssem[b2])
        for d in sc_d:
            if d is not None:
                d.wait()

    return gather_add


def _mlp_kernel(h_ref, a1_ref, m_ref, b1c_ref, g1_ref, bt1_ref, w2a_ref,
                b2c_ref, g2_ref, bt2_ref, wsel_ref, b3_ref,
                lo_ref, hi_ref):
    h = h_ref[...].astype(jnp.float32)
    d = jnp.dot(h, a1_ref[...], preferred_element_type=jnp.float32) + b1c_ref[...]
    v = jnp.dot(d * d, m_ref[...], preferred_element_type=jnp.float32)
    t = jnp.tanh(d * lax.rsqrt(v + 1e-5) * g1_ref[...] + bt1_ref[...])
    d2 = jnp.dot(t, w2a_ref[...], preferred_element_type=jnp.float32) + b2c_ref[...]
    v2 = jnp.dot(d2 * d2, m_ref[...], preferred_element_type=jnp.float32)
    t2 = jnp.tanh(d2 * lax.rsqrt(v2 + 1e-5) * g2_ref[...] + bt2_ref[...])
    s = lax.dot_general(wsel_ref[...], t2, (((1,), (1,)), ((), ())),
                        preferred_element_type=jnp.float32) + b3_ref[...]
    b = hi_ref.shape[-1]
    lo_ref[...] = s[0, :].reshape(1, 1, b)
    hi_ref[...] = s[1, :].reshape(1, 1, b)


def kernel(x, edge_index, W1, b1, g1, bt1, W2, b2, g2, bt2, W3, b3):
    N, D = x.shape
    E = edge_index.shape[1]
    H = W1.shape[1]
    H2 = 2 * H

    zh = jnp.zeros((D, H), jnp.float32)
    wa = jnp.block([[W1[:D], zh], [zh, W1[:D]]])
    wb = jnp.block([[W1[D:], zh], [zh, W1[D:]]])
    xp = x.reshape(N // 2, 2 * D)
    xs2, xe2 = pl.pallas_call(
        _proj_kernel,
        out_shape=[
            jax.ShapeDtypeStruct((N // 2, 2 * H), jnp.bfloat16),
            jax.ShapeDtypeStruct((N // 2, 2 * H), jnp.bfloat16),
        ],
    )(xp, wa, wb)
    xs = xs2.reshape(N, H)
    xe = xe2.reshape(N, H)

    eye = jnp.eye(H, dtype=jnp.float32)
    ones = jnp.full((H, H), 1.0 / H, dtype=jnp.float32)
    zero = jnp.zeros((H, H), dtype=jnp.float32)
    mblk = jnp.block([[ones, zero], [zero, ones]])
    a1 = jnp.block([[eye - ones, zero], [zero, eye - ones]])
    w2blk = jnp.block([[W2, zero], [zero, W2]])
    w2a = w2blk @ a1
    cat2 = lambda v: jnp.concatenate([v, v])[None, :]
    b1c = (cat2(b1) @ a1)
    b2c = (cat2(b2) @ a1)
    w3v = W3.reshape(1, H)
    zv = jnp.zeros((1, H), dtype=jnp.float32)
    wsel = jnp.block([[w3v, zv], [zv, w3v]])

    nslab = 5
    slab_rows = (E // 2) // nslab
    B = 6400
    nb = slab_rows // B
    full = lambda s: pl.BlockSpec(s, lambda i: (0, 0))
    mlp = pl.pallas_call(
        _mlp_kernel,
        grid=(nb,),
        in_specs=[
            pl.BlockSpec((B, H2), lambda i: (i, 0)),
            full((H2, H2)), full((H2, H2)), full((1, H2)), full((1, H2)),
            full((1, H2)), full((H2, H2)), full((1, H2)), full((1, H2)),
            full((1, H2)), full((2, H2)), full((1, 1)),
        ],
        out_specs=[
            pl.BlockSpec((1, 1, B), lambda i: (i, 0, 0)),
            pl.BlockSpec((1, 1, B), lambda i: (i, 0, 0)),
        ],
        out_shape=[
            jax.ShapeDtypeStruct((nb, 1, B), jnp.float32),
            jax.ShapeDtypeStruct((nb, 1, B), jnp.float32),
        ],
    )
    los, his = [], []
    for s in range(nslab):
        hp = _make_gather_add(E, H, slab_rows=slab_rows,
                              slab_base=s * slab_rows)(edge_index, xs, xe)
        lo, hi = mlp(hp, a1, mblk, b1c, cat2(g1), cat2(bt1), w2a, b2c,
                     cat2(g2), cat2(bt2), wsel, b3.reshape(1, 1))
        los.append(lo.reshape(slab_rows))
        his.append(hi.reshape(slab_rows))
    return jnp.concatenate(los + his)


# speedup vs baseline: 2.0196x; 2.0196x over previous
import functools

import jax
import jax.numpy as jnp
from jax import lax
from jax.experimental import pallas as pl
from jax.experimental.pallas import tpu as pltpu
from jax.experimental.pallas import tpu_sc as plsc

_NUM_CORES = 2
_NUM_SUBCORES = 16


def _proj_kernel(xp_ref, wa_ref, wb_ref, xs_ref, xe_ref):
    xv = xp_ref[...]
    xs_ref[...] = jnp.dot(xv, wa_ref[...], preferred_element_type=jnp.float32)
    xe_ref[...] = jnp.dot(xv, wb_ref[...], preferred_element_type=jnp.float32)


def _make_gather_add(E, H, slab_rows, slab_base):
    nw = _NUM_CORES * _NUM_SUBCORES
    half = E // 2
    rpw = slab_rows // nw
    unit = 200
    nu_side = rpw // unit
    nu = 2 * nu_side
    la = 2
    nbuf = 2 * la + 1

    mesh = plsc.VectorSubcoreMesh(
        core_axis_name="c", subcore_axis_name="s",
        num_cores=_NUM_CORES, num_subcores=_NUM_SUBCORES)

    @functools.partial(
        pl.kernel,
        out_type=jax.ShapeDtypeStruct((slab_rows, 2 * H), jnp.float32),
        mesh=mesh,
        scratch_types=[
            pltpu.VMEM((2 * rpw,), jnp.int32),
            pltpu.VMEM((2 * rpw,), jnp.int32),
        ] + [pltpu.VMEM((unit, H), jnp.float32) for _ in range(nbuf)]
          + [pltpu.SemaphoreType.DMA for _ in range(3 * nbuf + 1)],
        compiler_params=pltpu.CompilerParams(use_tc_tiling_on_sc=False),
    )
    def gather_add(eidx_hbm, xs_hbm, xe_hbm, out_hbm, idx_s, idx_e, *bufs):
        rows = bufs[:nbuf]
        isem = bufs[nbuf]
        bsem = bufs[nbuf + 1:2 * nbuf + 1]
        asem = bufs[2 * nbuf + 1:3 * nbuf + 1]
        ssem = bufs[3 * nbuf + 1:]
        wid = lax.axis_index("s") * _NUM_CORES + lax.axis_index("c")
        base = wid * rpw
        pre = []
        for row, dst in ((0, idx_s), (1, idx_e)):
            for side in range(2):
                e0 = side * half + slab_base + base
                pre.append(pltpu.async_copy(
                    eidx_hbm.at[row, pl.ds(e0, rpw)],
                    dst.at[pl.ds(side * rpw, rpw)], isem))
        for p in pre:
            p.wait()

        def src(u):
            side, j = u // nu_side, u % nu_side
            return side * rpw + j * unit

        def dst_rows(u):
            side, j = u // nu_side, u % nu_side
            return base + j * unit, side

        base_d = [None] * nbuf
        add_d = [None] * nbuf
        sc_d = [None] * nbuf
        for u in range(nu + 2 * la):
            if u < nu:
                b = u % nbuf
                if sc_d[b] is not None:
                    sc_d[b].wait()
                io = src(u)
                base_d[b] = pltpu.async_copy(
                    xs_hbm.at[idx_s.at[pl.ds(io, unit)]], rows[b], bsem[b])
            if la <= u < nu + la:
                b1 = (u - la) % nbuf
                base_d[b1].wait()
                io = src(u - la)
                add_d[b1] = pltpu.async_copy(
                    xe_hbm.at[idx_e.at[pl.ds(io, unit)]], rows[b1],
                    asem[b1], add=True)
            if u >= 2 * la:
                b2 = (u - 2 * la) % nbuf
                add_d[b2].wait()
                row, side = dst_rows(u - 2 * la)
                sc_d[b2] = pltpu.async_copy(
                    rows[b2],
                    out_hbm.at[pl.ds(row, unit), pl.ds(side * H, H)],
                    ssem[b2])
        for d in sc_d:
            if d is not None:
                d.wait()

    return gather_add


def _mlp_kernel(h_ref, a1_ref, m_ref, b1c_ref, g1_ref, bt1_ref, w2a_ref,
                b2c_ref, g2_ref, bt2_ref, wsel_ref, b3_ref,
                lo_ref, hi_ref):
    h = h_ref[...]
    d = jnp.dot(h, a1_ref[...], preferred_element_type=jnp.float32) + b1c_ref[...]
    v = jnp.dot(d * d, m_ref[...], preferred_element_type=jnp.float32)
    t = jnp.tanh(d * lax.rsqrt(v + 1e-5) * g1_ref[...] + bt1_ref[...])
    d2 = jnp.dot(t, w2a_ref[...], preferred_element_type=jnp.float32) + b2c_ref[...]
    v2 = jnp.dot(d2 * d2, m_ref[...], preferred_element_type=jnp.float32)
    t2 = jnp.tanh(d2 * lax.rsqrt(v2 + 1e-5) * g2_ref[...] + bt2_ref[...])
    s = lax.dot_general(wsel_ref[...], t2, (((1,), (1,)), ((), ())),
                        preferred_element_type=jnp.float32) + b3_ref[...]
    b = hi_ref.shape[-1]
    lo_ref[...] = s[0, :].reshape(1, 1, b)
    hi_ref[...] = s[1, :].reshape(1, 1, b)


def kernel(x, edge_index, W1, b1, g1, bt1, W2, b2, g2, bt2, W3, b3):
    N, D = x.shape
    E = edge_index.shape[1]
    H = W1.shape[1]
    H2 = 2 * H

    zh = jnp.zeros((D, H), jnp.float32)
    wa = jnp.block([[W1[:D], zh], [zh, W1[:D]]])
    wb = jnp.block([[W1[D:], zh], [zh, W1[D:]]])
    xp = x.reshape(N // 2, 2 * D)
    xs2, xe2 = pl.pallas_call(
        _proj_kernel,
        out_shape=[
            jax.ShapeDtypeStruct((N // 2, 2 * H), jnp.float32),
            jax.ShapeDtypeStruct((N // 2, 2 * H), jnp.float32),
        ],
    )(xp, wa, wb)
    xs = xs2.reshape(N, H)
    xe = xe2.reshape(N, H)

    eye = jnp.eye(H, dtype=jnp.float32)
    ones = jnp.full((H, H), 1.0 / H, dtype=jnp.float32)
    zero = jnp.zeros((H, H), dtype=jnp.float32)
    mblk = jnp.block([[ones, zero], [zero, ones]])
    a1 = jnp.block([[eye - ones, zero], [zero, eye - ones]])
    w2blk = jnp.block([[W2, zero], [zero, W2]])
    w2a = w2blk @ a1
    cat2 = lambda v: jnp.concatenate([v, v])[None, :]
    b1c = (cat2(b1) @ a1)
    b2c = (cat2(b2) @ a1)
    w3v = W3.reshape(1, H)
    zv = jnp.zeros((1, H), dtype=jnp.float32)
    wsel = jnp.block([[w3v, zv], [zv, w3v]])

    nslab = 5
    slab_rows = (E // 2) // nslab
    B = 6400
    nb = slab_rows // B
    full = lambda s: pl.BlockSpec(s, lambda i: (0, 0))
    mlp = pl.pallas_call(
        _mlp_kernel,
        grid=(nb,),
        in_specs=[
            pl.BlockSpec((B, H2), lambda i: (i, 0)),
            full((H2, H2)), full((H2, H2)), full((1, H2)), full((1, H2)),
            full((1, H2)), full((H2, H2)), full((1, H2)), full((1, H2)),
            full((1, H2)), full((2, H2)), full((1, 1)),
        ],
        out_specs=[
            pl.BlockSpec((1, 1, B), lambda i: (i, 0, 0)),
            pl.BlockSpec((1, 1, B), lambda i: (i, 0, 0)),
        ],
        out_shape=[
            jax.ShapeDtypeStruct((nb, 1, B), jnp.float32),
            jax.ShapeDtypeStruct((nb, 1, B), jnp.float32),
        ],
    )
    los, his = [], []
    for s in range(nslab):
        hp = _make_gather_add(E, H, slab_rows=slab_rows,
                              slab_base=s * slab_rows)(edge_index, xs, xe)
        lo, hi = mlp(hp, a1, mblk, b1c, cat2(g1), cat2(bt1), w2a, b2c,
                     cat2(g2), cat2(bt2), wsel, b3.reshape(1, 1))
        los.append(lo.reshape(slab_rows))
        his.append(hi.reshape(slab_rows))
    return jnp.concatenate(los + his)
